# baseline (speedup 1.0000x reference)
import jax
import jax.numpy as jnp
from jax.experimental import pallas as pl

POSITIONS = 200
BLOCK_ROWS = 2048


def _onehot_block(in_ref, out_ref):
    vals = in_ref[0, 0, :]
    cols = jax.lax.broadcasted_iota(jnp.int32, (BLOCK_ROWS, POSITIONS), 1)
    out_ref[...] = (vals[:, None] == cols).astype(jnp.float32)


def kernel(inputs):
    n = inputs.shape[0]
    grid = n // BLOCK_ROWS
    inputs3 = inputs.reshape(grid, 1, BLOCK_ROWS)
    return pl.pallas_call(
        _onehot_block,
        grid=(grid,),
        in_specs=[pl.BlockSpec((1, 1, BLOCK_ROWS), lambda i: (i, 0, 0))],
        out_specs=pl.BlockSpec((BLOCK_ROWS, POSITIONS), lambda i: (i, 0)),
        out_shape=jax.ShapeDtypeStruct((n, POSITIONS), jnp.float32),
    )(inputs3)
